# Initial kernel scaffold; baseline (speedup 1.0000x reference)
import jax
import jax.numpy as jnp
from jax.experimental import pallas as pl


def kernel(q, feat0, feat1, feat2, feat3, reference_points, spatial_shapes, level_start_index, q_proj_w, q_proj_b, value_proj_w, value_proj_b, samp_off_w, samp_off_b, attn_w_w, attn_w_b, output_proj_w, output_proj_b, out_proj_w, out_proj_b):
    raise NotImplementedError("write your pallas kernel here")


# speedup vs baseline: 22.9506x; 22.9506x over previous
import functools
import numpy as np
import jax
import jax.numpy as jnp
from jax import lax
from jax.experimental import pallas as pl
from jax.experimental.pallas import tpu as pltpu
from jax.experimental.pallas import tpu_sc as plsc

_D = 256
_H = 8
_L = 4
_P = 4
_DH = 32
_SHAPES = ((64, 64), (32, 32), (16, 16), (8, 8))
_STARTS = (0, 4096, 5120, 5376)
_LQ = 5440
_TQ = 160
_NQT = _LQ // _TQ
_B = 4
_NBLK = (_B * _LQ * _H) // 16
_NW = 32
_BPW = _NBLK // _NW


def _build_consts():
    K = _H * _L * _P
    kk = np.arange(K)
    lv = (kk // _P) % _L
    hv = kk // (_L * _P)
    wf = np.array([_SHAPES[l][1] for l in lv], np.float32)
    hf = np.array([_SHAPES[l][0] for l in lv], np.float32)
    wi = wf.astype(np.int32)
    sti = np.array([_STARTS[l] for l in lv], np.int32)
    hdi = hv.astype(np.int32)
    m8 = np.zeros((2 * _L, 2 * K), np.float32)
    cc = np.arange(2 * K)
    lv2 = ((cc // 2) // _P) % _L
    m8[lv2 * 2 + (cc % 2), cc] = 1.0
    inv = np.where(cc % 2 == 0,
                   1.0 / np.array([_SHAPES[l][1] for l in lv2], np.float32),
                   1.0 / np.array([_SHAPES[l][0] for l in lv2], np.float32))
    ex = np.zeros((2 * K, K), np.float32)
    ey = np.zeros((2 * K, K), np.float32)
    ex[2 * kk, kk] = 1.0
    ey[2 * kk + 1, kk] = 1.0
    s = (kk[:, None] // (_L * _P) == kk[None, :] // (_L * _P)).astype(np.float32)
    to2d = lambda a: jnp.asarray(a).reshape(1, -1)
    return (jnp.asarray(m8), to2d(inv), jnp.asarray(ex), jnp.asarray(ey),
            jnp.asarray(s), to2d(wf), to2d(hf), to2d(wi), to2d(sti), to2d(hdi))


def _proj_body(q_ref, val_ref, rp_ref, wq_ref, bq_ref, wv_ref, bv_ref,
               woff_ref, boff_ref, wa_ref, ba_ref,
               m8_ref, inv_ref, ex_ref, ey_ref, s_ref,
               wf_ref, hf_ref, wi_ref, sti_ref, hdi_ref,
               idx_ref, wgt_ref, vout_ref):
    b = pl.program_id(0)
    f32 = jnp.float32
    q = q_ref[0]
    vout_ref[0] = jnp.dot(val_ref[0], wv_ref[...],
                          preferred_element_type=f32, precision=lax.Precision.HIGHEST) + bv_ref[...]
    qp = jnp.dot(q, wq_ref[...], preferred_element_type=f32, precision=lax.Precision.HIGHEST) + bq_ref[...]
    off = jnp.dot(qp, woff_ref[...], preferred_element_type=f32, precision=lax.Precision.HIGHEST) + boff_ref[...]
    logits = jnp.dot(qp, wa_ref[...], preferred_element_type=f32, precision=lax.Precision.HIGHEST) + ba_ref[...]
    e = jnp.exp(logits)
    aw = e / jnp.dot(e, s_ref[...], preferred_element_type=f32, precision=lax.Precision.HIGHEST)
    loc = jnp.dot(rp_ref[0], m8_ref[...], preferred_element_type=f32, precision=lax.Precision.HIGHEST) \
        + off * inv_ref[...]
    wf = wf_ref[...]
    hf = hf_ref[...]
    x = jnp.dot(loc, ex_ref[...], preferred_element_type=f32, precision=lax.Precision.HIGHEST) * wf - 0.5
    y = jnp.dot(loc, ey_ref[...], preferred_element_type=f32, precision=lax.Precision.HIGHEST) * hf - 0.5
    x0 = jnp.floor(x)
    y0 = jnp.floor(y)
    wx1 = x - x0
    wx0 = 1.0 - wx1
    wy1 = y - y0
    wy0 = 1.0 - wy1
    base = hdi_ref[...] * _LQ + sti_ref[...] + b * (_H * _LQ)
    wi = wi_ref[...]
    corners = ((x0, y0, wx0 * wy0), (x0 + 1.0, y0, wx1 * wy0),
               (x0, y0 + 1.0, wx0 * wy1), (x0 + 1.0, y0 + 1.0, wx1 * wy1))
    K = _H * _L * _P
    for ci, (xf, yf, wgt) in enumerate(corners):
        valid = (xf >= 0.0) & (xf <= wf - 1.0) & (yf >= 0.0) & (yf <= hf - 1.0)
        xc = jnp.clip(xf, 0.0, wf - 1.0).astype(jnp.int32)
        yc = jnp.clip(yf, 0.0, hf - 1.0).astype(jnp.int32)
        idx_ref[0, :, ci * K:(ci + 1) * K] = base + yc * wi + xc
        wgt_ref[0, :, ci * K:(ci + 1) * K] = wgt * aw * valid.astype(f32)


def _out_body(x_ref, w1_ref, b1_ref, w2_ref, b2_ref, o_ref):
    f32 = jnp.float32
    t = jnp.dot(x_ref[0], w1_ref[...], preferred_element_type=f32, precision=lax.Precision.HIGHEST) + b1_ref[...]
    o_ref[0] = jnp.dot(t, w2_ref[...], preferred_element_type=f32, precision=lax.Precision.HIGHEST) + b2_ref[...]


def _sc_body(table_ref, idx_ref, wgt_ref, out_ref, idx_v, rows_v, w_v, acc_v, sem):
    wid = lax.axis_index("s") * 2 + lax.axis_index("c")
    lanes = lax.iota(jnp.int32, 16)

    def blk_body(i, carry):
        blk = wid * _BPW + i
        pltpu.sync_copy(idx_ref.at[blk], idx_v)
        pltpu.sync_copy(wgt_ref.at[blk], w_v)
        cps = [pltpu.async_copy(table_ref.at[idx_v.at[j]],
                                rows_v.at[pl.ds(j * 128, 128)], sem)
               for j in range(8)]
        for c in cps:
            c.wait()

        def s_body(s, accs):
            wv = w_v[s]
            r = lanes * 64 + s
            out = []
            for ch in range(_DH):
                cv = jnp.full((16,), ch, jnp.int32)
                val = plsc.load_gather(rows_v, [r, cv])
                out.append(accs[ch] + wv * val)
            return tuple(out)

        accs = lax.fori_loop(
            0, 64, s_body,
            tuple(jnp.zeros((16,), jnp.float32) for _ in range(_DH)))
        for ch in range(_DH):
            acc_v[ch] = accs[ch]
        pltpu.sync_copy(acc_v, out_ref.at[blk])
        return carry

    lax.fori_loop(0, _BPW, blk_body, 0)


@functools.cache
def _sc_sample():
    return pl.kernel(
        _sc_body,
        out_type=jax.ShapeDtypeStruct((_NBLK, _DH, 16), jnp.float32),
        mesh=plsc.VectorSubcoreMesh(core_axis_name="c", subcore_axis_name="s"),
        compiler_params=pltpu.CompilerParams(needs_layout_passes=False,
                                             use_tc_tiling_on_sc=False),
        scratch_types=[
            pltpu.VMEM((8, 128), jnp.int32),
            pltpu.VMEM((1024, _DH), jnp.float32),
            pltpu.VMEM((64, 16), jnp.float32),
            pltpu.VMEM((_DH, 16), jnp.float32),
            pltpu.SemaphoreType.DMA,
        ],
    )


def kernel(q, feat0, feat1, feat2, feat3, reference_points, spatial_shapes,
           level_start_index, q_proj_w, q_proj_b, value_proj_w, value_proj_b,
           samp_off_w, samp_off_b, attn_w_w, attn_w_b, output_proj_w,
           output_proj_b, out_proj_w, out_proj_b):
    B = q.shape[0]
    feats = (feat0, feat1, feat2, feat3)
    value = jnp.concatenate(
        [f.reshape(B, _D, -1).transpose(0, 2, 1) for f in feats], axis=1)
    rp8 = reference_points.reshape(B, _LQ, 2 * _L)
    consts = _build_consts()
    to2d = lambda a: a.reshape(1, -1)
    K = _H * _L * _P

    full = lambda shape: pl.BlockSpec(shape, lambda b, i: (0,) * len(shape))
    tile = lambda w: pl.BlockSpec((1, _TQ, w), lambda b, i: (b, i, 0))
    idx, wgt, v = pl.pallas_call(
        _proj_body,
        grid=(B, _NQT),
        in_specs=[
            tile(_D), tile(_D), tile(2 * _L),
            full((_D, _D)), full((1, _D)), full((_D, _D)), full((1, _D)),
            full((_D, 2 * K)), full((1, 2 * K)), full((_D, K)), full((1, K)),
            full((2 * _L, 2 * K)), full((1, 2 * K)),
            full((2 * K, K)), full((2 * K, K)), full((K, K)),
            full((1, K)), full((1, K)), full((1, K)), full((1, K)),
            full((1, K)),
        ],
        out_specs=[tile(4 * K), tile(4 * K), tile(_D)],
        out_shape=[
            jax.ShapeDtypeStruct((B, _LQ, 4 * K), jnp.int32),
            jax.ShapeDtypeStruct((B, _LQ, 4 * K), jnp.float32),
            jax.ShapeDtypeStruct((B, _LQ, _D), jnp.float32),
        ],
    )(q, value, rp8, q_proj_w, to2d(q_proj_b), value_proj_w, to2d(value_proj_b),
      samp_off_w, to2d(samp_off_b), attn_w_w, to2d(attn_w_b), *consts)

    table = v.reshape(B, _LQ, _H, _DH).transpose(0, 2, 1, 3) \
             .reshape(B * _H * _LQ, _DH)
    blocks = lambda a: a.reshape(B, _LQ, 4, _H, _L * _P) \
                        .transpose(0, 1, 3, 2, 4).reshape(_NBLK, 16, 64)
    idx_b = blocks(idx).reshape(_NBLK, 8, 128)
    wgt_b = blocks(wgt).transpose(0, 2, 1)

    sc_out = _sc_sample()(table, idx_b, wgt_b)
    core = sc_out.transpose(0, 2, 1).reshape(B, _LQ, _D)

    out = pl.pallas_call(
        _out_body,
        grid=(B, _NQT),
        in_specs=[tile(_D), full((_D, _D)), full((1, _D)),
                  full((_D, _D)), full((1, _D))],
        out_specs=tile(_D),
        out_shape=jax.ShapeDtypeStruct((B, _LQ, _D), jnp.float32),
    )(core, output_proj_w, to2d(output_proj_b), out_proj_w, to2d(out_proj_b))
    return out
